# baseline (speedup 1.0000x reference)
import functools

import jax
import jax.numpy as jnp
from jax.experimental import pallas as pl
from jax.experimental.pallas import tpu as pltpu


def _fused_body(y_ref, x_ref, wc_ref, w_ref, out_ref,
                m_ref, s_ref, p_ref, cl_ref,
                *, cutoffs, block_n, n_blocks):
    j = pl.program_id(0)
    y = y_ref[...]
    c1, c2, c3 = cutoffs[1], cutoffs[2], cutoffs[3]
    l = jnp.where(y < c1, 0, jnp.where(y < c2, c1, c2))
    r = jnp.where(y < c1, c1, jnp.where(y < c2, c2, c3))

    @pl.when(j == 0)
    def _init():
        m_ref[...] = jnp.full_like(m_ref, -1e30)
        s_ref[...] = jnp.zeros_like(s_ref)
        p_ref[...] = jnp.zeros_like(p_ref)
        ccl = jnp.dot(x_ref[...], wc_ref[...],
                      preferred_element_type=jnp.float32)
        mm = jnp.max(ccl, axis=1, keepdims=True)
        lse = mm + jnp.log(jnp.sum(jnp.exp(ccl - mm), axis=1, keepdims=True))
        pick = jnp.where(y < c1, ccl[:, 0:1],
                         jnp.where(y < c2, ccl[:, 1:2], ccl[:, 2:3]))
        cl_ref[...] = pick - lse

    z = jnp.dot(x_ref[...], w_ref[...],
                preferred_element_type=jnp.float32)
    cols = j * block_n + jax.lax.broadcasted_iota(jnp.int32, (1, block_n), 1)
    mask = (cols >= l) & (cols < r)
    zm = jnp.where(mask, z, -1e30)
    bm = jnp.max(zm, axis=1, keepdims=True)
    m_old = m_ref[...]
    m_new = jnp.maximum(m_old, bm)
    e = jnp.where(mask, jnp.exp(z - m_new), 0.0)
    s_ref[...] = s_ref[...] * jnp.exp(m_old - m_new) + jnp.sum(e, axis=1, keepdims=True)
    m_ref[...] = m_new
    p_ref[...] = p_ref[...] + jnp.sum(jnp.where(cols == y, z, 0.0), axis=1,
                                      keepdims=True)

    @pl.when(j == n_blocks - 1)
    def _fin():
        out_ref[...] = -cl_ref[...] - (p_ref[...] - (m_ref[...] + jnp.log(s_ref[...])))


def _fused_nll(x, y, wc_t, logits, cutoffs, block_n, interpret=False):
    n_tok, hidden = x.shape
    vocab = logits.shape[1]
    n_blocks = pl.cdiv(vocab, block_n)
    ncl = wc_t.shape[1]
    y2d = y.reshape(n_tok, 1)
    out = pl.pallas_call(
        functools.partial(_fused_body, cutoffs=cutoffs, block_n=block_n,
                          n_blocks=n_blocks),
        grid=(n_blocks,),
        in_specs=[
            pl.BlockSpec((n_tok, 1), lambda j: (0, 0)),
            pl.BlockSpec((n_tok, hidden), lambda j: (0, 0)),
            pl.BlockSpec((hidden, ncl), lambda j: (0, 0)),
            pl.BlockSpec((hidden, block_n), lambda j: (0, j)),
        ],
        out_specs=pl.BlockSpec((n_tok, 1), lambda j: (0, 0)),
        out_shape=jax.ShapeDtypeStruct((n_tok, 1), jnp.float32),
        scratch_shapes=[pltpu.VMEM((n_tok, 1), jnp.float32)] * 4,
        compiler_params=pltpu.CompilerParams(
            dimension_semantics=("arbitrary",)),
        interpret=interpret,
    )(y2d, x, wc_t, logits)
    return out[:, 0]


def kernel(x, y, W_cluster, logits):
    return _fused_nll(x, y, W_cluster.T, logits,
                      cutoffs=(0, 20000, 60000, 100000), block_n=512)
